# baseline (speedup 1.0000x reference)
import functools

import jax
import jax.numpy as jnp
import numpy as np
from jax.experimental import pallas as pl
from jax.experimental.pallas import tpu as pltpu
from jax.experimental.pallas import tpu_sc as plsc

_EPS = 1e-5
_BLK_S = 2048


def _ln_body(x_ref, pos_ref, gamma_ref, beta_ref, out_ref):
    h = x_ref[0] + pos_ref[...]
    mean = jnp.mean(h, axis=-1, keepdims=True)
    c = h - mean
    var = jnp.mean(c * c, axis=-1, keepdims=True)
    inv = jax.lax.rsqrt(var + _EPS)
    out_ref[0] = c * inv * gamma_ref[...] + beta_ref[...]


@jax.jit
def _ln_tc(x, pos_table, gamma, beta):
    b, s, d = x.shape
    grid = (s // _BLK_S, b)
    return pl.pallas_call(
        _ln_body,
        grid=grid,
        in_specs=[
            pl.BlockSpec((1, _BLK_S, d), lambda i, j: (j, i, 0)),
            pl.BlockSpec((_BLK_S, d), lambda i, j: (i, 0)),
            pl.BlockSpec((1, d), lambda i, j: (0, 0)),
            pl.BlockSpec((1, d), lambda i, j: (0, 0)),
        ],
        out_specs=pl.BlockSpec((1, _BLK_S, d), lambda i, j: (j, i, 0)),
        out_shape=jax.ShapeDtypeStruct((b, s, d), x.dtype),
        compiler_params=pltpu.CompilerParams(
            vmem_limit_bytes=100 * 1024 * 1024),
    )(x, pos_table, gamma.reshape(1, d), beta.reshape(1, d))


_NC = 2
_NS = 16
_L = 16
_R = 32


def _xlane_sum(v):
    dnums = jax.lax.GatherDimensionNumbers(
        offset_dims=(), collapsed_slice_dims=(0,), start_index_map=(0,))
    lane = jax.lax.iota(jnp.int32, _L)
    for k in (1, 2, 4, 8):
        idx = jnp.reshape(lane ^ k, (_L, 1))
        v = v + jax.lax.gather(
            v, idx, dnums, slice_sizes=(1,),
            mode=jax.lax.GatherScatterMode.PROMISE_IN_BOUNDS)
    return v


def _newton_rsqrt(v):
    i = jax.lax.bitcast_convert_type(v, jnp.int32)
    i = jnp.int32(0x5F3759DF) - jax.lax.shift_right_logical(i, 1)
    y = jax.lax.bitcast_convert_type(i, jnp.float32)
    for _ in range(3):
        y = y * (1.5 - 0.5 * v * y * y)
    return y


def _sc_body(x_hbm, pos_hbm, gamma_hbm, beta_hbm, out_hbm,
             xbuf, obuf, pbuf, gbuf, bbuf, sx, sp, so):
    d = gbuf.shape[0]
    nj = d // _L
    n_rows, _ = x_hbm.shape
    n_seq, _ = pos_hbm.shape
    batch = n_rows // n_seq
    seq_per_tile = n_seq // (_NC * _NS)
    n_chunks = seq_per_tile // _R
    n_steps = n_chunks * batch

    cid = jax.lax.axis_index("c")
    sid = jax.lax.axis_index("s")
    wid = sid * _NC + cid
    seq_base = wid * seq_per_tile

    pltpu.sync_copy(gamma_hbm, gbuf)
    pltpu.sync_copy(beta_hbm, bbuf)

    def x_row0(s):
        return (s % batch) * n_seq + seq_base + (s // batch) * _R

    def start_x(s):
        p = jax.lax.rem(s, 2)
        pltpu.make_async_copy(
            x_hbm.at[pl.ds(x_row0(s), _R)],
            xbuf.at[pl.ds(p * _R, _R)], sx.at[p]).start()

    def start_pos(c):
        pltpu.make_async_copy(
            pos_hbm.at[pl.ds(seq_base + c * _R, _R)], pbuf, sp.at[0]).start()

    start_x(jnp.int32(0))
    start_pos(jnp.int32(0))

    def step_body(s, carry):
        p = jax.lax.rem(s, 2)
        c = s // batch
        b = jax.lax.rem(s, batch)

        @pl.when(s + 1 < n_steps)
        def _():
            start_x(s + 1)

        pltpu.make_async_copy(
            x_hbm.at[pl.ds(0, _R)], xbuf.at[pl.ds(p * _R, _R)],
            sx.at[p]).wait()

        @pl.when(b == 0)
        def _():
            pltpu.make_async_copy(
                x_hbm.at[pl.ds(0, _R)], pbuf, sp.at[0]).wait()

        @pl.when(s >= 2)
        def _():
            pltpu.make_async_copy(
                x_hbm.at[pl.ds(0, _R)], obuf.at[pl.ds(p * _R, _R)],
                so.at[p]).wait()

        xb = p * _R

        def row_body(r, rcarry):
            acc = jnp.zeros((_L,), jnp.float32)
            acc2 = jnp.zeros((_L,), jnp.float32)
            for j in range(nj):
                sl = pl.ds(j * _L, _L)
                v = xbuf[xb + r, sl] + pbuf[r, sl]
                obuf[xb + r, sl] = v
                acc = acc + v
                acc2 = acc2 + v * v
            mean_v = _xlane_sum(acc) * (1.0 / d)
            ex2_v = _xlane_sum(acc2) * (1.0 / d)
            var_v = ex2_v - mean_v * mean_v + _EPS
            rstd = _newton_rsqrt(var_v)
            scale = rstd
            shift = mean_v * rstd
            for j in range(nj):
                sl = pl.ds(j * _L, _L)
                obuf[xb + r, sl] = (
                    (obuf[xb + r, sl] * scale - shift) * gbuf[sl] + bbuf[sl])
            return rcarry

        jax.lax.fori_loop(0, _R, row_body, 0, unroll=2)

        pltpu.make_async_copy(
            obuf.at[pl.ds(xb, _R)],
            out_hbm.at[pl.ds(x_row0(s), _R)], so.at[p]).start()

        @pl.when(jnp.logical_and(b == batch - 1, c + 1 < n_chunks))
        def _():
            start_pos(c + 1)
        return carry

    jax.lax.fori_loop(0, n_steps, step_body, 0, unroll=False)

    for p in range(2):
        pltpu.make_async_copy(
            x_hbm.at[pl.ds(0, _R)], obuf.at[pl.ds(p * _R, _R)],
            so.at[p]).wait()


@jax.jit
def _ln_sc(x, pos_table, gamma, beta):
    b, s, d = x.shape
    x2 = x.reshape(b * s, d)
    mesh = plsc.VectorSubcoreMesh(core_axis_name="c", subcore_axis_name="s")
    out = pl.kernel(
        _sc_body,
        out_type=jax.ShapeDtypeStruct((b * s, d), x.dtype),
        mesh=mesh,
        scratch_types=[
            pltpu.VMEM((2 * _R, d), jnp.float32),
            pltpu.VMEM((2 * _R, d), jnp.float32),
            pltpu.VMEM((_R, d), jnp.float32),
            pltpu.VMEM((d,), jnp.float32),
            pltpu.VMEM((d,), jnp.float32),
            pltpu.SemaphoreType.DMA((2,)),
            pltpu.SemaphoreType.DMA((2,)),
            pltpu.SemaphoreType.DMA((2,)),
        ],
    )(x2, pos_table, gamma, beta)
    return out.reshape(b, s, d)


def kernel(x, pos_table, gamma, beta, batch_size):
    return _ln_sc(x, pos_table, gamma, beta)


# speedup vs baseline: 2.7646x; 2.7646x over previous
import functools

import jax
import jax.numpy as jnp
import numpy as np
from jax.experimental import pallas as pl
from jax.experimental.pallas import tpu as pltpu
from jax.experimental.pallas import tpu_sc as plsc

_EPS = 1e-5
_BLK_S = 2048


def _ln_body(x_ref, pos_ref, gamma_ref, beta_ref, out_ref):
    h = x_ref[0] + pos_ref[...]
    mean = jnp.mean(h, axis=-1, keepdims=True)
    c = h - mean
    var = jnp.mean(c * c, axis=-1, keepdims=True)
    inv = jax.lax.rsqrt(var + _EPS)
    out_ref[0] = c * inv * gamma_ref[...] + beta_ref[...]


@jax.jit
def _ln_tc(x, pos_table, gamma, beta):
    b, s, d = x.shape
    grid = (s // _BLK_S, b)
    return pl.pallas_call(
        _ln_body,
        grid=grid,
        in_specs=[
            pl.BlockSpec((1, _BLK_S, d), lambda i, j: (j, i, 0)),
            pl.BlockSpec((_BLK_S, d), lambda i, j: (i, 0)),
            pl.BlockSpec((1, d), lambda i, j: (0, 0)),
            pl.BlockSpec((1, d), lambda i, j: (0, 0)),
        ],
        out_specs=pl.BlockSpec((1, _BLK_S, d), lambda i, j: (j, i, 0)),
        out_shape=jax.ShapeDtypeStruct((b, s, d), x.dtype),
        compiler_params=pltpu.CompilerParams(
            vmem_limit_bytes=100 * 1024 * 1024),
    )(x, pos_table, gamma.reshape(1, d), beta.reshape(1, d))


_NC = 2
_NS = 16
_L = 16
_R = 32


def _xlane_sum(v):
    dnums = jax.lax.GatherDimensionNumbers(
        offset_dims=(), collapsed_slice_dims=(0,), start_index_map=(0,))
    lane = jax.lax.iota(jnp.int32, _L)
    for k in (1, 2, 4, 8):
        idx = jnp.reshape(lane ^ k, (_L, 1))
        v = v + jax.lax.gather(
            v, idx, dnums, slice_sizes=(1,),
            mode=jax.lax.GatherScatterMode.PROMISE_IN_BOUNDS)
    return v


def _newton_rsqrt(v):
    i = jax.lax.bitcast_convert_type(v, jnp.int32)
    i = jnp.int32(0x5F3759DF) - jax.lax.shift_right_logical(i, 1)
    y = jax.lax.bitcast_convert_type(i, jnp.float32)
    for _ in range(3):
        y = y * (1.5 - 0.5 * v * y * y)
    return y


def _sc_body(x_hbm, pos_hbm, gamma_hbm, beta_hbm, out_hbm,
             xbuf, obuf, pbuf, gbuf, bbuf, sx, sp, so, base_row=0):
    d = gbuf.shape[0]
    nj = d // _L
    n_rows, _ = out_hbm.shape
    n_seq, _ = pos_hbm.shape
    batch = n_rows // n_seq
    seq_per_tile = n_seq // (_NC * _NS)
    n_chunks = seq_per_tile // _R
    n_steps = n_chunks * batch

    cid = jax.lax.axis_index("c")
    sid = jax.lax.axis_index("s")
    wid = sid * _NC + cid
    seq_base = wid * seq_per_tile

    pltpu.sync_copy(gamma_hbm, gbuf)
    pltpu.sync_copy(beta_hbm, bbuf)

    def x_row0(s):
        return (s % batch) * n_seq + seq_base + (s // batch) * _R

    def start_x(s):
        p = jax.lax.rem(s, 2)
        pltpu.make_async_copy(
            x_hbm.at[pl.ds(base_row + x_row0(s), _R)],
            xbuf.at[pl.ds(p * _R, _R)], sx.at[p]).start()

    def start_pos(c):
        pltpu.make_async_copy(
            pos_hbm.at[pl.ds(seq_base + c * _R, _R)], pbuf, sp.at[0]).start()

    start_x(jnp.int32(0))
    start_pos(jnp.int32(0))

    def step_body(s, carry):
        p = jax.lax.rem(s, 2)
        c = s // batch
        b = jax.lax.rem(s, batch)

        @pl.when(s + 1 < n_steps)
        def _():
            start_x(s + 1)

        pltpu.make_async_copy(
            x_hbm.at[pl.ds(0, _R)], xbuf.at[pl.ds(p * _R, _R)],
            sx.at[p]).wait()

        @pl.when(b == 0)
        def _():
            pltpu.make_async_copy(
                x_hbm.at[pl.ds(0, _R)], pbuf, sp.at[0]).wait()

        @pl.when(s >= 2)
        def _():
            pltpu.make_async_copy(
                x_hbm.at[pl.ds(0, _R)], obuf.at[pl.ds(p * _R, _R)],
                so.at[p]).wait()

        xb = p * _R

        def row_body(r, rcarry):
            acc = jnp.zeros((_L,), jnp.float32)
            acc2 = jnp.zeros((_L,), jnp.float32)
            hs = []
            for j in range(nj):
                sl = pl.ds(j * _L, _L)
                v = xbuf[xb + r, sl] + pbuf[r, sl]
                acc = acc + v
                acc2 = acc2 + v * v
                hs.append(v)
            mean_v = _xlane_sum(acc) * (1.0 / d)
            ex2_v = _xlane_sum(acc2) * (1.0 / d)
            var_v = ex2_v - mean_v * mean_v + _EPS
            rstd = _newton_rsqrt(var_v)
            for j in range(nj):
                sl = pl.ds(j * _L, _L)
                obuf[xb + r, sl] = (
                    (hs[j] - mean_v) * rstd * gbuf[sl] + bbuf[sl])
            return rcarry

        jax.lax.fori_loop(0, _R, row_body, 0, unroll=1)

        pltpu.make_async_copy(
            obuf.at[pl.ds(xb, _R)],
            out_hbm.at[pl.ds(x_row0(s), _R)], so.at[p]).start()

        @pl.when(jnp.logical_and(b == batch - 1, c + 1 < n_chunks))
        def _():
            start_pos(c + 1)
        return carry

    jax.lax.fori_loop(0, n_steps, step_body, 0, unroll=False)

    for p in range(2):
        pltpu.make_async_copy(
            x_hbm.at[pl.ds(0, _R)], obuf.at[pl.ds(p * _R, _R)],
            so.at[p]).wait()


@jax.jit
def _ln_sc(x, pos_table, gamma, beta):
    b, s, d = x.shape
    x2 = x.reshape(b * s, d)
    mesh = plsc.VectorSubcoreMesh(core_axis_name="c", subcore_axis_name="s")
    out = pl.kernel(
        _sc_body,
        out_type=jax.ShapeDtypeStruct((b * s, d), x.dtype),
        mesh=mesh,
        scratch_types=[
            pltpu.VMEM((2 * _R, d), jnp.float32),
            pltpu.VMEM((2 * _R, d), jnp.float32),
            pltpu.VMEM((_R, d), jnp.float32),
            pltpu.VMEM((d,), jnp.float32),
            pltpu.VMEM((d,), jnp.float32),
            pltpu.SemaphoreType.DMA((2,)),
            pltpu.SemaphoreType.DMA((2,)),
            pltpu.SemaphoreType.DMA((2,)),
        ],
    )(x2, pos_table, gamma, beta)
    return out.reshape(b, s, d)


@jax.jit
def _ln_hybrid(x, pos_table, gamma, beta):
    b, s, d = x.shape
    b_tc = b - 1
    tc_out = pl.pallas_call(
        _ln_body,
        grid=(s // _BLK_S, b_tc),
        in_specs=[
            pl.BlockSpec((1, _BLK_S, d), lambda i, j: (j, i, 0)),
            pl.BlockSpec((_BLK_S, d), lambda i, j: (i, 0)),
            pl.BlockSpec((1, d), lambda i, j: (0, 0)),
            pl.BlockSpec((1, d), lambda i, j: (0, 0)),
        ],
        out_specs=pl.BlockSpec((1, _BLK_S, d), lambda i, j: (j, i, 0)),
        out_shape=jax.ShapeDtypeStruct((b_tc, s, d), x.dtype),
        compiler_params=pltpu.CompilerParams(
            vmem_limit_bytes=100 * 1024 * 1024),
    )(x, pos_table, gamma.reshape(1, d), beta.reshape(1, d))
    mesh = plsc.VectorSubcoreMesh(core_axis_name="c", subcore_axis_name="s")
    sc_out = pl.kernel(
        functools.partial(_sc_body, base_row=b_tc * s),
        out_type=jax.ShapeDtypeStruct((s, d), x.dtype),
        mesh=mesh,
        scratch_types=[
            pltpu.VMEM((2 * _R, d), jnp.float32),
            pltpu.VMEM((2 * _R, d), jnp.float32),
            pltpu.VMEM((_R, d), jnp.float32),
            pltpu.VMEM((d,), jnp.float32),
            pltpu.VMEM((d,), jnp.float32),
            pltpu.SemaphoreType.DMA((2,)),
            pltpu.SemaphoreType.DMA((2,)),
            pltpu.SemaphoreType.DMA((2,)),
        ],
    )(x.reshape(b * s, d), pos_table, gamma, beta)
    return jnp.concatenate([tc_out, sc_out[None]], axis=0)


def kernel(x, pos_table, gamma, beta, batch_size):
    return _ln_hybrid(x, pos_table, gamma, beta)


# speedup vs baseline: 2.7712x; 1.0024x over previous
import functools

import jax
import jax.numpy as jnp
import numpy as np
from jax.experimental import pallas as pl
from jax.experimental.pallas import tpu as pltpu
from jax.experimental.pallas import tpu_sc as plsc

_EPS = 1e-5
_BLK_S = 2048


def _ln_body(x_ref, pos_ref, gamma_ref, beta_ref, out_ref):
    h = x_ref[0] + pos_ref[...]
    mean = jnp.mean(h, axis=-1, keepdims=True)
    c = h - mean
    var = jnp.mean(c * c, axis=-1, keepdims=True)
    inv = jax.lax.rsqrt(var + _EPS)
    out_ref[0] = c * inv * gamma_ref[...] + beta_ref[...]


@jax.jit
def _ln_tc(x, pos_table, gamma, beta):
    b, s, d = x.shape
    grid = (s // _BLK_S, b)
    return pl.pallas_call(
        _ln_body,
        grid=grid,
        in_specs=[
            pl.BlockSpec((1, _BLK_S, d), lambda i, j: (j, i, 0)),
            pl.BlockSpec((_BLK_S, d), lambda i, j: (i, 0)),
            pl.BlockSpec((1, d), lambda i, j: (0, 0)),
            pl.BlockSpec((1, d), lambda i, j: (0, 0)),
        ],
        out_specs=pl.BlockSpec((1, _BLK_S, d), lambda i, j: (j, i, 0)),
        out_shape=jax.ShapeDtypeStruct((b, s, d), x.dtype),
        compiler_params=pltpu.CompilerParams(
            vmem_limit_bytes=100 * 1024 * 1024),
    )(x, pos_table, gamma.reshape(1, d), beta.reshape(1, d))


_NC = 2
_NS = 16
_L = 16
_R = 32


def _xlane_sum(v):
    dnums = jax.lax.GatherDimensionNumbers(
        offset_dims=(), collapsed_slice_dims=(0,), start_index_map=(0,))
    lane = jax.lax.iota(jnp.int32, _L)
    for k in (1, 2, 4, 8):
        idx = jnp.reshape(lane ^ k, (_L, 1))
        v = v + jax.lax.gather(
            v, idx, dnums, slice_sizes=(1,),
            mode=jax.lax.GatherScatterMode.PROMISE_IN_BOUNDS)
    return v


def _newton_rsqrt(v):
    i = jax.lax.bitcast_convert_type(v, jnp.int32)
    i = jnp.int32(0x5F3759DF) - jax.lax.shift_right_logical(i, 1)
    y = jax.lax.bitcast_convert_type(i, jnp.float32)
    for _ in range(3):
        y = y * (1.5 - 0.5 * v * y * y)
    return y


def _sc_body(x_hbm, pos_hbm, gamma_hbm, beta_hbm, out_hbm,
             xbuf, obuf, pbuf, gbuf, bbuf, sx, sp, so, base_row=0):
    d = gbuf.shape[0]
    nj = d // _L
    n_rows, _ = out_hbm.shape
    n_seq, _ = pos_hbm.shape
    batch = n_rows // n_seq
    seq_per_tile = n_seq // (_NC * _NS)
    n_chunks = seq_per_tile // _R
    n_steps = n_chunks * batch

    cid = jax.lax.axis_index("c")
    sid = jax.lax.axis_index("s")
    wid = sid * _NC + cid
    seq_base = wid * seq_per_tile

    pltpu.sync_copy(gamma_hbm, gbuf)
    pltpu.sync_copy(beta_hbm, bbuf)

    def x_row0(s):
        return (s % batch) * n_seq + seq_base + (s // batch) * _R

    def start_x(s):
        p = jax.lax.rem(s, 2)
        pltpu.make_async_copy(
            x_hbm.at[pl.ds(base_row + x_row0(s), _R)],
            xbuf.at[pl.ds(p * _R, _R)], sx.at[p]).start()

    def start_pos(c):
        pltpu.make_async_copy(
            pos_hbm.at[pl.ds(seq_base + c * _R, _R)], pbuf, sp.at[0]).start()

    start_x(jnp.int32(0))
    start_pos(jnp.int32(0))

    def step_body(s, carry):
        p = jax.lax.rem(s, 2)
        c = s // batch
        b = jax.lax.rem(s, batch)

        @pl.when(s + 1 < n_steps)
        def _():
            start_x(s + 1)

        pltpu.make_async_copy(
            x_hbm.at[pl.ds(0, _R)], xbuf.at[pl.ds(p * _R, _R)],
            sx.at[p]).wait()

        @pl.when(b == 0)
        def _():
            pltpu.make_async_copy(
                x_hbm.at[pl.ds(0, _R)], pbuf, sp.at[0]).wait()

        @pl.when(s >= 2)
        def _():
            pltpu.make_async_copy(
                x_hbm.at[pl.ds(0, _R)], obuf.at[pl.ds(p * _R, _R)],
                so.at[p]).wait()

        xb = p * _R

        def row_body(r, rcarry):
            acc = jnp.zeros((_L,), jnp.float32)
            acc2 = jnp.zeros((_L,), jnp.float32)
            hs = []
            for j in range(nj):
                sl = pl.ds(j * _L, _L)
                v = xbuf[xb + r, sl] + pbuf[r, sl]
                acc = acc + v
                acc2 = acc2 + v * v
                hs.append(v)
            mean_v = _xlane_sum(acc) * (1.0 / d)
            ex2_v = _xlane_sum(acc2) * (1.0 / d)
            var_v = ex2_v - mean_v * mean_v + _EPS
            rstd = _newton_rsqrt(var_v)
            for j in range(nj):
                sl = pl.ds(j * _L, _L)
                obuf[xb + r, sl] = (
                    (hs[j] - mean_v) * rstd * gbuf[sl] + bbuf[sl])
            return rcarry

        jax.lax.fori_loop(0, _R, row_body, 0, unroll=1)

        pltpu.make_async_copy(
            obuf.at[pl.ds(xb, _R)],
            out_hbm.at[pl.ds(x_row0(s), _R)], so.at[p]).start()

        @pl.when(jnp.logical_and(b == batch - 1, c + 1 < n_chunks))
        def _():
            start_pos(c + 1)
        return carry

    jax.lax.fori_loop(0, n_steps, step_body, 0, unroll=False)

    for p in range(2):
        pltpu.make_async_copy(
            x_hbm.at[pl.ds(0, _R)], obuf.at[pl.ds(p * _R, _R)],
            so.at[p]).wait()


@jax.jit
def _ln_sc(x, pos_table, gamma, beta):
    b, s, d = x.shape
    x2 = x.reshape(b * s, d)
    mesh = plsc.VectorSubcoreMesh(core_axis_name="c", subcore_axis_name="s")
    out = pl.kernel(
        _sc_body,
        out_type=jax.ShapeDtypeStruct((b * s, d), x.dtype),
        mesh=mesh,
        scratch_types=[
            pltpu.VMEM((2 * _R, d), jnp.float32),
            pltpu.VMEM((2 * _R, d), jnp.float32),
            pltpu.VMEM((_R, d), jnp.float32),
            pltpu.VMEM((d,), jnp.float32),
            pltpu.VMEM((d,), jnp.float32),
            pltpu.SemaphoreType.DMA((2,)),
            pltpu.SemaphoreType.DMA((2,)),
            pltpu.SemaphoreType.DMA((2,)),
        ],
    )(x2, pos_table, gamma, beta)
    return out.reshape(b, s, d)


@jax.jit
def _ln_hybrid(x, pos_table, gamma, beta):
    b, s, d = x.shape
    b_tc = b - 1
    mesh = plsc.VectorSubcoreMesh(core_axis_name="c", subcore_axis_name="s")
    sc_out = pl.kernel(
        functools.partial(_sc_body, base_row=b_tc * s),
        out_type=jax.ShapeDtypeStruct((s, d), x.dtype),
        mesh=mesh,
        scratch_types=[
            pltpu.VMEM((2 * _R, d), jnp.float32),
            pltpu.VMEM((2 * _R, d), jnp.float32),
            pltpu.VMEM((_R, d), jnp.float32),
            pltpu.VMEM((d,), jnp.float32),
            pltpu.VMEM((d,), jnp.float32),
            pltpu.SemaphoreType.DMA((2,)),
            pltpu.SemaphoreType.DMA((2,)),
            pltpu.SemaphoreType.DMA((2,)),
        ],
    )(x.reshape(b * s, d), pos_table, gamma, beta)
    tc_out = pl.pallas_call(
        _ln_body,
        grid=(s // _BLK_S, b_tc),
        in_specs=[
            pl.BlockSpec((1, _BLK_S, d), lambda i, j: (j, i, 0)),
            pl.BlockSpec((_BLK_S, d), lambda i, j: (i, 0)),
            pl.BlockSpec((1, d), lambda i, j: (0, 0)),
            pl.BlockSpec((1, d), lambda i, j: (0, 0)),
        ],
        out_specs=pl.BlockSpec((1, _BLK_S, d), lambda i, j: (j, i, 0)),
        out_shape=jax.ShapeDtypeStruct((b_tc, s, d), x.dtype),
        compiler_params=pltpu.CompilerParams(
            vmem_limit_bytes=100 * 1024 * 1024),
    )(x, pos_table, gamma.reshape(1, d), beta.reshape(1, d))
    return jnp.concatenate([tc_out, sc_out[None]], axis=0)


def kernel(x, pos_table, gamma, beta, batch_size):
    return _ln_hybrid(x, pos_table, gamma, beta)


# speedup vs baseline: 7.4037x; 2.6716x over previous
import jax
import jax.numpy as jnp
from jax.experimental import pallas as pl
from jax.experimental.pallas import tpu as pltpu

_EPS = 1e-5
_BLK_S = 2048


def _ln_body(x_ref, pos_ref, gamma_ref, beta_ref, out_ref):
    h = x_ref[0] + pos_ref[...]
    mean = jnp.mean(h, axis=-1, keepdims=True)
    c = h - mean
    var = jnp.mean(c * c, axis=-1, keepdims=True)
    inv = jax.lax.rsqrt(var + _EPS)
    out_ref[0] = c * inv * gamma_ref[...] + beta_ref[...]


@jax.jit
def _ln_tc(x, pos_table, gamma, beta):
    b, s, d = x.shape
    grid = (s // _BLK_S, b)
    return pl.pallas_call(
        _ln_body,
        grid=grid,
        in_specs=[
            pl.BlockSpec((1, _BLK_S, d), lambda i, j: (j, i, 0)),
            pl.BlockSpec((_BLK_S, d), lambda i, j: (i, 0)),
            pl.BlockSpec((1, d), lambda i, j: (0, 0)),
            pl.BlockSpec((1, d), lambda i, j: (0, 0)),
        ],
        out_specs=pl.BlockSpec((1, _BLK_S, d), lambda i, j: (j, i, 0)),
        out_shape=jax.ShapeDtypeStruct((b, s, d), x.dtype),
        compiler_params=pltpu.CompilerParams(
            vmem_limit_bytes=100 * 1024 * 1024),
    )(x, pos_table, gamma.reshape(1, d), beta.reshape(1, d))


def kernel(x, pos_table, gamma, beta, batch_size):
    return _ln_tc(x, pos_table, gamma, beta)
